# baseline (speedup 1.0000x reference)
import functools

import jax
import jax.numpy as jnp
from jax import lax
from jax.experimental import pallas as pl
from jax.experimental.pallas import tpu as pltpu
from jax.experimental.pallas import tpu_sc as plsc

B = 16384
D_OUT = 12
L = 16

_TABLE_SHAPES = ((7, 3), (12, 3), (3, 2), (24, 4))
_COL_OFFSETS = (0, 3, 6, 8)


def _make_kernel(ncores=None):
    info = plsc.get_sparse_core_info()
    if ncores is None:
        ncores = info.num_cores
    nw = ncores * info.num_subcores
    b_per_w = B // nw
    n_blocks = b_per_w // L

    mesh = plsc.VectorSubcoreMesh(core_axis_name="c", subcore_axis_name="s",
                                  num_cores=ncores)

    @functools.partial(
        pl.kernel,
        mesh=mesh,
        out_type=jax.ShapeDtypeStruct((B * D_OUT,), jnp.float32),
        compiler_params=pltpu.CompilerParams(needs_layout_passes=False),
        scratch_types=[
            pltpu.VMEM((_TABLE_SHAPES[0][0] * _TABLE_SHAPES[0][1],), jnp.float32),
            pltpu.VMEM((_TABLE_SHAPES[1][0] * _TABLE_SHAPES[1][1],), jnp.float32),
            pltpu.VMEM((_TABLE_SHAPES[2][0] * _TABLE_SHAPES[2][1],), jnp.float32),
            pltpu.VMEM((_TABLE_SHAPES[3][0] * _TABLE_SHAPES[3][1],), jnp.float32),
            pltpu.VMEM((b_per_w,), jnp.int32),
            pltpu.VMEM((b_per_w,), jnp.int32),
            pltpu.VMEM((b_per_w,), jnp.int32),
            pltpu.VMEM((b_per_w,), jnp.int32),
            pltpu.VMEM((b_per_w * D_OUT,), jnp.float32),
            pltpu.SemaphoreType.DMA,
            pltpu.SemaphoreType.DMA,
        ],
    )
    def run(wd_hbm, mo_hbm, lk_hbm, st_hbm,
            ww_hbm, wm_hbm, wl_hbm, ws_hbm,
            out_hbm,
            ww_v, wm_v, wl_v, ws_v,
            wd_v, mo_v, lk_v, st_v,
            out_v, sem, out_sem):
        wid = lax.axis_index("s") * ncores + lax.axis_index("c")
        base = wid * b_per_w

        copies = [
            pltpu.async_copy(wd_hbm.at[pl.ds(base, b_per_w)], wd_v, sem),
            pltpu.async_copy(mo_hbm.at[pl.ds(base, b_per_w)], mo_v, sem),
            pltpu.async_copy(lk_hbm.at[pl.ds(base, b_per_w)], lk_v, sem),
            pltpu.async_copy(st_hbm.at[pl.ds(base, b_per_w)], st_v, sem),
            pltpu.async_copy(ww_hbm, ww_v, sem),
            pltpu.async_copy(wm_hbm, wm_v, sem),
            pltpu.async_copy(wl_hbm, wl_v, sem),
            pltpu.async_copy(ws_hbm, ws_v, sem),
        ]
        for c in copies:
            c.wait()

        iota12 = lax.iota(jnp.int32, L) * D_OUT
        ones = [jnp.full((L,), c, jnp.int32) for c in range(D_OUT)]

        def block(i, carry):
            off = i * L
            rowbase = off * D_OUT + iota12
            idxs = (wd_v[pl.ds(off, L)], mo_v[pl.ds(off, L)],
                    lk_v[pl.ds(off, L)], st_v[pl.ds(off, L)])
            for tab, idx, (_, width), coff in zip(
                    (ww_v, wm_v, wl_v, ws_v), idxs, _TABLE_SHAPES, _COL_OFFSETS):
                scaled = idx * width
                for c in range(width):
                    val = plsc.load_gather(tab, [scaled + ones[c]])
                    plsc.store_scatter(out_v, [rowbase + ones[coff + c]], val)
            return carry

        n_q = 2
        blocks_per_q = n_blocks // n_q
        q_elems = b_per_w * D_OUT // n_q
        out_copies = []
        for q in range(n_q):
            lax.fori_loop(q * blocks_per_q, (q + 1) * blocks_per_q, block, 0)
            out_copies.append(pltpu.async_copy(
                out_v.at[pl.ds(q * q_elems, q_elems)],
                out_hbm.at[pl.ds(base * D_OUT + q * q_elems, q_elems)],
                out_sem))
        for c in out_copies:
            c.wait()

    return run


_sc_embed = _make_kernel()


def kernel(weekday, month, leak_type, start_time,
           W_weekday, W_month, W_leak_type, W_start_time):
    flat = _sc_embed(
        weekday.astype(jnp.int32), month.astype(jnp.int32),
        leak_type.astype(jnp.int32), start_time.astype(jnp.int32),
        W_weekday.reshape(-1), W_month.reshape(-1),
        W_leak_type.reshape(-1), W_start_time.reshape(-1))
    return flat.reshape(B, D_OUT)


# speedup vs baseline: 1.0023x; 1.0023x over previous
import functools

import jax
import jax.numpy as jnp
from jax import lax
from jax.experimental import pallas as pl
from jax.experimental.pallas import tpu as pltpu
from jax.experimental.pallas import tpu_sc as plsc

B = 16384
D_OUT = 12
L = 16

_TABLE_SHAPES = ((7, 3), (12, 3), (3, 2), (24, 4))
_COL_OFFSETS = (0, 3, 6, 8)


def _make_kernel(ncores=None):
    info = plsc.get_sparse_core_info()
    if ncores is None:
        ncores = info.num_cores
    nw = ncores * info.num_subcores
    b_per_w = B // nw
    n_blocks = b_per_w // L

    mesh = plsc.VectorSubcoreMesh(core_axis_name="c", subcore_axis_name="s",
                                  num_cores=ncores)

    @functools.partial(
        pl.kernel,
        mesh=mesh,
        out_type=jax.ShapeDtypeStruct((B * D_OUT,), jnp.float32),
        compiler_params=pltpu.CompilerParams(needs_layout_passes=False),
        scratch_types=[
            pltpu.VMEM((_TABLE_SHAPES[0][0] * _TABLE_SHAPES[0][1],), jnp.float32),
            pltpu.VMEM((_TABLE_SHAPES[1][0] * _TABLE_SHAPES[1][1],), jnp.float32),
            pltpu.VMEM((_TABLE_SHAPES[2][0] * _TABLE_SHAPES[2][1],), jnp.float32),
            pltpu.VMEM((_TABLE_SHAPES[3][0] * _TABLE_SHAPES[3][1],), jnp.float32),
            pltpu.VMEM((b_per_w,), jnp.int32),
            pltpu.VMEM((b_per_w,), jnp.int32),
            pltpu.VMEM((b_per_w,), jnp.int32),
            pltpu.VMEM((b_per_w,), jnp.int32),
            pltpu.VMEM((b_per_w * D_OUT,), jnp.float32),
            pltpu.SemaphoreType.DMA,
            pltpu.SemaphoreType.DMA,
            pltpu.SemaphoreType.DMA,
        ],
    )
    def run(wd_hbm, mo_hbm, lk_hbm, st_hbm,
            ww_hbm, wm_hbm, wl_hbm, ws_hbm,
            out_hbm,
            ww_v, wm_v, wl_v, ws_v,
            wd_v, mo_v, lk_v, st_v,
            out_v, sem, sem_b, out_sem):
        wid = lax.axis_index("s") * ncores + lax.axis_index("c")
        base = wid * b_per_w
        half = b_per_w // 2

        idx_pairs = ((wd_hbm, wd_v), (mo_hbm, mo_v),
                     (lk_hbm, lk_v), (st_hbm, st_v))
        copies_a = [
            pltpu.async_copy(h.at[pl.ds(base, half)], v.at[pl.ds(0, half)],
                             sem)
            for h, v in idx_pairs
        ] + [
            pltpu.async_copy(ww_hbm, ww_v, sem),
            pltpu.async_copy(wm_hbm, wm_v, sem),
            pltpu.async_copy(wl_hbm, wl_v, sem),
            pltpu.async_copy(ws_hbm, ws_v, sem),
        ]
        copies_b = [
            pltpu.async_copy(h.at[pl.ds(base + half, half)],
                             v.at[pl.ds(half, half)], sem_b)
            for h, v in idx_pairs
        ]

        iota12 = lax.iota(jnp.int32, L) * D_OUT
        ones = [jnp.full((L,), c, jnp.int32) for c in range(D_OUT)]

        def block(i, carry):
            off = i * L
            rowbase = off * D_OUT + iota12
            idxs = (wd_v[pl.ds(off, L)], mo_v[pl.ds(off, L)],
                    lk_v[pl.ds(off, L)], st_v[pl.ds(off, L)])
            for tab, idx, (_, width), coff in zip(
                    (ww_v, wm_v, wl_v, ws_v), idxs, _TABLE_SHAPES, _COL_OFFSETS):
                scaled = idx * width
                for c in range(width):
                    val = plsc.load_gather(tab, [scaled + ones[c]])
                    plsc.store_scatter(out_v, [rowbase + ones[coff + c]], val)
            return carry

        blocks_per_q = n_blocks // 2
        q_elems = b_per_w * D_OUT // 2
        out_copies = []
        for q, staged in enumerate((copies_a, copies_b)):
            for c in staged:
                c.wait()
            lax.fori_loop(q * blocks_per_q, (q + 1) * blocks_per_q, block, 0)
            out_copies.append(pltpu.async_copy(
                out_v.at[pl.ds(q * q_elems, q_elems)],
                out_hbm.at[pl.ds(base * D_OUT + q * q_elems, q_elems)],
                out_sem))
        for c in out_copies:
            c.wait()

    return run


_sc_embed = _make_kernel()


def kernel(weekday, month, leak_type, start_time,
           W_weekday, W_month, W_leak_type, W_start_time):
    flat = _sc_embed(
        weekday.astype(jnp.int32), month.astype(jnp.int32),
        leak_type.astype(jnp.int32), start_time.astype(jnp.int32),
        W_weekday.reshape(-1), W_month.reshape(-1),
        W_leak_type.reshape(-1), W_start_time.reshape(-1))
    return flat.reshape(B, D_OUT)
